# baseline (device time: 179934 ns/iter reference)
import jax
import jax.numpy as jnp
from jax import lax
from jax.experimental import pallas as pl
from jax.experimental.pallas import tpu as pltpu

N_DEV = 4
K_TAPS = 4


def kernel(x, k, Wp):
    B, S, C = x.shape
    C_out = Wp.shape[1]
    S2 = S // 2
    S4 = S // 4
    f32 = jnp.float32
    bf16 = jnp.bfloat16

    wp_bf = Wp.astype(bf16)

    def body(k_ref, wp_ref, x_ref, out_ref,
             xbuf, comm_cw, comm_ccw,
             rs_send, rs_recv, ag_send, ag_recv, ld_sems):
        my = lax.axis_index("i")
        left = lax.rem(my + N_DEV - 1, N_DEV)
        right = lax.rem(my + 1, N_DEV)
        order = [my, left, right, lax.rem(my + 2, N_DEV)]

        barrier = pltpu.get_barrier_semaphore()
        for nbr in (left, right):
            pl.semaphore_signal(
                barrier, inc=1,
                device_id=(nbr,), device_id_type=pl.DeviceIdType.MESH,
            )

        kv = k_ref[...].astype(bf16)

        def start_load(i):
            cp = pltpu.make_async_copy(
                x_ref.at[pl.ds(order[i], 1)],
                xbuf.at[pl.ds(i % 2, 1)],
                ld_sems.at[i % 2],
            )
            cp.start()
            return cp

        loads = {0: start_load(0), 1: start_load(1)}

        def compute_half(i, half, first):
            if first:
                loads[i].wait()
            if half == 0:
                pad = jnp.concatenate(
                    [jnp.zeros((K_TAPS - 1, C), bf16),
                     xbuf[i % 2, 0:S2].astype(bf16)], axis=0
                )
            else:
                pad = xbuf[i % 2, S2 - (K_TAPS - 1):S].astype(bf16)
            acc = pad[0:S2] * kv[0]
            for t in range(1, K_TAPS):
                acc = acc + pad[t:t + S2] * kv[t]
            a = acc * jax.nn.sigmoid(acc)
            res = jnp.dot(
                a, wp_ref[...], preferred_element_type=f32
            ).astype(bf16)
            out_ref[pl.ds(order[i], 1), pl.ds(half * S2, S2)] = res[None]

        def rs_copy(dir_idx, t, sub, c_send, comm, nbr):
            row = dir_idx * S2 + sub * S4
            return pltpu.make_async_remote_copy(
                src_ref=out_ref.at[pl.ds(c_send, 1), pl.ds(row, S4)],
                dst_ref=comm.at[pl.ds(t, 1), pl.ds(sub * S4, S4)],
                send_sem=rs_send.at[dir_idx, t, sub],
                recv_sem=rs_recv.at[dir_idx, t, sub],
                device_id=(nbr,),
                device_id_type=pl.DeviceIdType.MESH,
            )

        def rs_add(dir_idx, t, sub, c_recv, comm):
            row = dir_idx * S2 + sub * S4
            sl = (pl.ds(c_recv, 1), pl.ds(row, S4))
            out_ref[sl] = (
                out_ref[sl] + comm[pl.ds(t, 1), pl.ds(sub * S4, S4)]
            )

        def ag_copy(dir_idx, t, sub, c_send, nbr):
            row = dir_idx * S2 + sub * S4
            sl = (pl.ds(c_send, 1), pl.ds(row, S4))
            return pltpu.make_async_remote_copy(
                src_ref=out_ref.at[sl],
                dst_ref=out_ref.at[sl],
                send_sem=ag_send.at[dir_idx, t, sub],
                recv_sem=ag_recv.at[dir_idx, t, sub],
                device_id=(nbr,),
                device_id_type=pl.DeviceIdType.MESH,
            )

        cw_send = [order[0], order[1], order[3]]
        cw_recv = [order[1], order[3], order[2]]
        ccw_send = [order[0], order[2], order[3]]
        ccw_recv = [order[2], order[3], order[1]]

        rs = {}

        def rs_start(dir_idx, t, sub):
            comm, nbr = ((comm_cw, right) if dir_idx == 0
                         else (comm_ccw, left))
            c = cw_send[t] if dir_idx == 0 else ccw_send[t]
            rs[(dir_idx, t, sub)] = rs_copy(dir_idx, t, sub, c, comm, nbr)
            rs[(dir_idx, t, sub)].start()

        def rs_finish(dir_idx, t, sub):
            comm = comm_cw if dir_idx == 0 else comm_ccw
            c = cw_recv[t] if dir_idx == 0 else ccw_recv[t]
            rs[(dir_idx, t, sub)].wait()
            rs_add(dir_idx, t, sub, c, comm)

        compute_half(0, 0, True)
        pl.semaphore_wait(barrier, 2)
        rs_start(0, 0, 0)
        rs_start(0, 0, 1)
        compute_half(0, 1, False)
        rs_start(1, 0, 0)
        rs_start(1, 0, 1)
        loads[2] = start_load(2)
        compute_half(1, 0, True)
        rs_finish(0, 0, 0)
        rs_start(0, 1, 0)
        rs_finish(0, 0, 1)
        rs_start(0, 1, 1)
        compute_half(1, 1, False)
        loads[3] = start_load(3)
        compute_half(2, 1, True)
        rs_finish(1, 0, 0)
        rs_start(1, 1, 0)
        rs_finish(1, 0, 1)
        rs_start(1, 1, 1)
        compute_half(3, 0, True)
        rs_finish(0, 1, 0)
        rs_start(0, 2, 0)
        rs_finish(0, 1, 1)
        rs_start(0, 2, 1)
        compute_half(3, 1, False)
        rs_finish(1, 1, 0)
        rs_start(1, 2, 0)
        rs_finish(1, 1, 1)
        rs_start(1, 2, 1)
        compute_half(2, 0, False)

        ag_cw_chunks = [order[2], order[0], order[1]]
        ag_ccw_chunks = [order[1], order[0], order[2]]
        ag = {}
        for sub in range(2):
            rs_finish(0, 2, sub)
            ag[(0, 0, sub)] = ag_copy(0, 0, sub, ag_cw_chunks[0], right)
            ag[(0, 0, sub)].start()
            rs_finish(1, 2, sub)
            ag[(1, 0, sub)] = ag_copy(1, 0, sub, ag_ccw_chunks[0], left)
            ag[(1, 0, sub)].start()
        for t in range(1, N_DEV - 1):
            for sub in range(2):
                ag[(0, t - 1, sub)].wait()
                ag[(0, t, sub)] = ag_copy(0, t, sub, ag_cw_chunks[t], right)
                ag[(0, t, sub)].start()
                ag[(1, t - 1, sub)].wait()
                ag[(1, t, sub)] = ag_copy(1, t, sub, ag_ccw_chunks[t], left)
                ag[(1, t, sub)].start()
        for sub in range(2):
            ag[(0, 2, sub)].wait()
            ag[(1, 2, sub)].wait()

    out = pl.pallas_call(
        body,
        out_shape=jax.ShapeDtypeStruct((B, S, C_out), bf16),
        in_specs=[
            pl.BlockSpec(memory_space=pltpu.VMEM),
            pl.BlockSpec(memory_space=pltpu.VMEM),
            pl.BlockSpec(memory_space=pl.ANY),
        ],
        out_specs=pl.BlockSpec(memory_space=pltpu.VMEM),
        scratch_shapes=[
            pltpu.VMEM((2, S, C), f32),
            pltpu.VMEM((N_DEV - 1, S2, C_out), bf16),
            pltpu.VMEM((N_DEV - 1, S2, C_out), bf16),
            pltpu.SemaphoreType.DMA((2, N_DEV - 1, 2)),
            pltpu.SemaphoreType.DMA((2, N_DEV - 1, 2)),
            pltpu.SemaphoreType.DMA((2, N_DEV - 1, 2)),
            pltpu.SemaphoreType.DMA((2, N_DEV - 1, 2)),
            pltpu.SemaphoreType.DMA((2,)),
        ],
        compiler_params=pltpu.CompilerParams(
            collective_id=0,
            vmem_limit_bytes=100 * 1024 * 1024,
        ),
    )(k, wp_bf, x)
    return out


# device time: 175870 ns/iter; 1.0231x vs baseline; 1.0231x over previous
import jax
import jax.numpy as jnp
from jax import lax
from jax.experimental import pallas as pl
from jax.experimental.pallas import tpu as pltpu

N_DEV = 4
K_TAPS = 4


def kernel(x, k, Wp):
    B, S, C = x.shape
    C_out = Wp.shape[1]
    H = C_out // 2
    S2 = S // 2
    f32 = jnp.float32
    bf16 = jnp.bfloat16

    wp_bf = Wp.astype(bf16)

    def body(k_ref, wp_ref, x_ref, out_ref,
             xbuf, comm_cw, comm_ccw,
             rs_send, rs_recv, ag_send, ag_recv, ld_sems):
        my = lax.axis_index("i")
        left = lax.rem(my + N_DEV - 1, N_DEV)
        right = lax.rem(my + 1, N_DEV)
        order = [my, left, right, lax.rem(my + 2, N_DEV)]

        barrier = pltpu.get_barrier_semaphore()
        for nbr in (left, right):
            pl.semaphore_signal(
                barrier, inc=1,
                device_id=(nbr,), device_id_type=pl.DeviceIdType.MESH,
            )

        kv = k_ref[...].astype(bf16)

        def start_load(i):
            cp = pltpu.make_async_copy(
                x_ref.at[pl.ds(order[i], 1)],
                xbuf.at[pl.ds(i % 2, 1)],
                ld_sems.at[i % 2],
            )
            cp.start()
            return cp

        loads = {0: start_load(0), 1: start_load(1)}

        def compute_half(i, half):
            if half == 0:
                loads[i].wait()
                pad = jnp.concatenate(
                    [jnp.zeros((K_TAPS - 1, C), bf16),
                     xbuf[i % 2, 0:S2].astype(bf16)], axis=0
                )
            else:
                pad = xbuf[i % 2, S2 - (K_TAPS - 1):S].astype(bf16)
            acc = pad[0:S2] * kv[0]
            for t in range(1, K_TAPS):
                acc = acc + pad[t:t + S2] * kv[t]
            a = acc * jax.nn.sigmoid(acc)
            res = jnp.dot(
                a, wp_ref[...], preferred_element_type=f32
            ).astype(bf16)
            out_ref[pl.ds(order[i], 1), pl.ds(half * S2, S2)] = res[None]

        def rs_copy(dir_idx, t, sub, c_send, comm, nbr):
            sl = (pl.ds(c_send, 1), pl.ds(sub * S2, S2),
                  pl.ds(dir_idx * H, H))
            return pltpu.make_async_remote_copy(
                src_ref=out_ref.at[sl],
                dst_ref=comm.at[pl.ds(t, 1), pl.ds(sub * S2, S2)],
                send_sem=rs_send.at[dir_idx, t, sub],
                recv_sem=rs_recv.at[dir_idx, t, sub],
                device_id=(nbr,),
                device_id_type=pl.DeviceIdType.MESH,
            )

        def rs_add(dir_idx, t, sub, c_recv, comm):
            sl = (pl.ds(c_recv, 1), pl.ds(sub * S2, S2),
                  pl.ds(dir_idx * H, H))
            out_ref[sl] = (
                out_ref[sl]
                + comm[pl.ds(t, 1), pl.ds(sub * S2, S2)]
            )

        def ag_copy(dir_idx, t, sub, c_send, nbr):
            sl = (pl.ds(c_send, 1), pl.ds(sub * S2, S2),
                  pl.ds(dir_idx * H, H))
            return pltpu.make_async_remote_copy(
                src_ref=out_ref.at[sl],
                dst_ref=out_ref.at[sl],
                send_sem=ag_send.at[dir_idx, t, sub],
                recv_sem=ag_recv.at[dir_idx, t, sub],
                device_id=(nbr,),
                device_id_type=pl.DeviceIdType.MESH,
            )

        cw_send = [order[0], order[1], order[3]]
        cw_recv = [order[1], order[3], order[2]]
        ccw_send = [order[0], order[2], order[3]]
        ccw_recv = [order[2], order[3], order[1]]

        rs = {}

        def rs_start(dir_idx, t, sub):
            comm, nbr = ((comm_cw, right) if dir_idx == 0
                         else (comm_ccw, left))
            c = cw_send[t] if dir_idx == 0 else ccw_send[t]
            rs[(dir_idx, t, sub)] = rs_copy(dir_idx, t, sub, c, comm, nbr)
            rs[(dir_idx, t, sub)].start()

        def rs_finish(dir_idx, t, sub):
            comm = comm_cw if dir_idx == 0 else comm_ccw
            c = cw_recv[t] if dir_idx == 0 else ccw_recv[t]
            rs[(dir_idx, t, sub)].wait()
            rs_add(dir_idx, t, sub, c, comm)

        compute_half(0, 0)
        pl.semaphore_wait(barrier, 2)
        rs_start(0, 0, 0)
        rs_start(1, 0, 0)
        compute_half(0, 1)
        rs_start(0, 0, 1)
        rs_start(1, 0, 1)
        loads[2] = start_load(2)
        compute_half(1, 0)
        rs_finish(0, 0, 0)
        rs_start(0, 1, 0)
        compute_half(1, 1)
        rs_finish(0, 0, 1)
        rs_start(0, 1, 1)
        loads[3] = start_load(3)
        compute_half(2, 0)
        rs_finish(1, 0, 0)
        rs_start(1, 1, 0)
        compute_half(2, 1)
        rs_finish(1, 0, 1)
        rs_start(1, 1, 1)
        compute_half(3, 0)
        rs_finish(0, 1, 0)
        rs_start(0, 2, 0)
        rs_finish(1, 1, 0)
        rs_start(1, 2, 0)
        compute_half(3, 1)
        rs_finish(0, 1, 1)
        rs_start(0, 2, 1)
        rs_finish(1, 1, 1)
        rs_start(1, 2, 1)

        ag_cw_chunks = [order[2], order[0], order[1]]
        ag_ccw_chunks = [order[1], order[0], order[2]]
        ag = {}
        for sub in range(2):
            rs_finish(0, 2, sub)
            ag[(0, 0, sub)] = ag_copy(0, 0, sub, ag_cw_chunks[0], right)
            ag[(0, 0, sub)].start()
            rs_finish(1, 2, sub)
            ag[(1, 0, sub)] = ag_copy(1, 0, sub, ag_ccw_chunks[0], left)
            ag[(1, 0, sub)].start()
        for t in range(1, N_DEV - 1):
            for sub in range(2):
                ag[(0, t - 1, sub)].wait()
                ag[(0, t, sub)] = ag_copy(0, t, sub, ag_cw_chunks[t], right)
                ag[(0, t, sub)].start()
                ag[(1, t - 1, sub)].wait()
                ag[(1, t, sub)] = ag_copy(1, t, sub, ag_ccw_chunks[t], left)
                ag[(1, t, sub)].start()
        for sub in range(2):
            ag[(0, 2, sub)].wait()
            ag[(1, 2, sub)].wait()

    out = pl.pallas_call(
        body,
        out_shape=jax.ShapeDtypeStruct((B, S, C_out), bf16),
        in_specs=[
            pl.BlockSpec(memory_space=pltpu.VMEM),
            pl.BlockSpec(memory_space=pltpu.VMEM),
            pl.BlockSpec(memory_space=pl.ANY),
        ],
        out_specs=pl.BlockSpec(memory_space=pltpu.VMEM),
        scratch_shapes=[
            pltpu.VMEM((2, S, C), f32),
            pltpu.VMEM((N_DEV - 1, S, H), bf16),
            pltpu.VMEM((N_DEV - 1, S, H), bf16),
            pltpu.SemaphoreType.DMA((2, N_DEV - 1, 2)),
            pltpu.SemaphoreType.DMA((2, N_DEV - 1, 2)),
            pltpu.SemaphoreType.DMA((2, N_DEV - 1, 2)),
            pltpu.SemaphoreType.DMA((2, N_DEV - 1, 2)),
            pltpu.SemaphoreType.DMA((2,)),
        ],
        compiler_params=pltpu.CompilerParams(
            collective_id=0,
            vmem_limit_bytes=100 * 1024 * 1024,
        ),
    )(k, wp_bf, x)
    return out


# device time: 172512 ns/iter; 1.0430x vs baseline; 1.0195x over previous
import jax
import jax.numpy as jnp
from jax import lax
from jax.experimental import pallas as pl
from jax.experimental.pallas import tpu as pltpu

N_DEV = 4
K_TAPS = 4


def kernel(x, k, Wp):
    B, S, C = x.shape
    C_out = Wp.shape[1]
    H = C_out // 2
    S2 = S // 2
    f32 = jnp.float32
    bf16 = jnp.bfloat16

    def body(k_ref, wp_ref, x_ref, out_ref,
             xbuf, comm_cw, comm_ccw,
             rs_send, rs_recv, ag_send, ag_recv, ld_sems):
        my = lax.axis_index("i")
        left = lax.rem(my + N_DEV - 1, N_DEV)
        right = lax.rem(my + 1, N_DEV)
        order = [my, left, right, lax.rem(my + 2, N_DEV)]

        barrier = pltpu.get_barrier_semaphore()
        for nbr in (left, right):
            pl.semaphore_signal(
                barrier, inc=1,
                device_id=(nbr,), device_id_type=pl.DeviceIdType.MESH,
            )

        kv = k_ref[...].astype(bf16)
        wp_bf = wp_ref[...].astype(bf16)

        def start_load(i):
            cp = pltpu.make_async_copy(
                x_ref.at[pl.ds(order[i], 1)],
                xbuf.at[pl.ds(i % 2, 1)],
                ld_sems.at[i % 2],
            )
            cp.start()
            return cp

        loads = {0: start_load(0), 1: start_load(1)}

        def compute_half(i, half):
            if half == 0:
                loads[i].wait()
                pad = jnp.concatenate(
                    [jnp.zeros((K_TAPS - 1, C), bf16),
                     xbuf[i % 2, 0:S2].astype(bf16)], axis=0
                )
            else:
                pad = xbuf[i % 2, S2 - (K_TAPS - 1):S].astype(bf16)
            acc = pad[0:S2] * kv[0]
            for t in range(1, K_TAPS):
                acc = acc + pad[t:t + S2] * kv[t]
            a = acc * jax.nn.sigmoid(acc)
            res = jnp.dot(
                a, wp_bf, preferred_element_type=f32
            ).astype(bf16)
            out_ref[pl.ds(order[i], 1), pl.ds(half * S2, S2)] = res[None]

        def rs_copy(dir_idx, t, sub, c_send, comm, nbr):
            sl = (pl.ds(c_send, 1), pl.ds(sub * S2, S2),
                  pl.ds(dir_idx * H, H))
            return pltpu.make_async_remote_copy(
                src_ref=out_ref.at[sl],
                dst_ref=comm.at[pl.ds(t, 1), pl.ds(sub * S2, S2)],
                send_sem=rs_send.at[dir_idx, t, sub],
                recv_sem=rs_recv.at[dir_idx, t, sub],
                device_id=(nbr,),
                device_id_type=pl.DeviceIdType.MESH,
            )

        def rs_add(dir_idx, t, sub, c_recv, comm):
            sl = (pl.ds(c_recv, 1), pl.ds(sub * S2, S2),
                  pl.ds(dir_idx * H, H))
            out_ref[sl] = (
                out_ref[sl]
                + comm[pl.ds(t, 1), pl.ds(sub * S2, S2)]
            )

        def ag_copy(dir_idx, t, sub, c_send, nbr):
            sl = (pl.ds(c_send, 1), pl.ds(sub * S2, S2),
                  pl.ds(dir_idx * H, H))
            return pltpu.make_async_remote_copy(
                src_ref=out_ref.at[sl],
                dst_ref=out_ref.at[sl],
                send_sem=ag_send.at[dir_idx, t, sub],
                recv_sem=ag_recv.at[dir_idx, t, sub],
                device_id=(nbr,),
                device_id_type=pl.DeviceIdType.MESH,
            )

        cw_send = [order[0], order[1], order[3]]
        cw_recv = [order[1], order[3], order[2]]
        ccw_send = [order[0], order[2], order[3]]
        ccw_recv = [order[2], order[3], order[1]]

        rs = {}

        def rs_start(dir_idx, t, sub):
            comm, nbr = ((comm_cw, right) if dir_idx == 0
                         else (comm_ccw, left))
            c = cw_send[t] if dir_idx == 0 else ccw_send[t]
            rs[(dir_idx, t, sub)] = rs_copy(dir_idx, t, sub, c, comm, nbr)
            rs[(dir_idx, t, sub)].start()

        def rs_finish(dir_idx, t, sub):
            comm = comm_cw if dir_idx == 0 else comm_ccw
            c = cw_recv[t] if dir_idx == 0 else ccw_recv[t]
            rs[(dir_idx, t, sub)].wait()
            rs_add(dir_idx, t, sub, c, comm)

        compute_half(0, 0)
        pl.semaphore_wait(barrier, 2)
        rs_start(0, 0, 0)
        rs_start(1, 0, 0)
        compute_half(0, 1)
        rs_start(0, 0, 1)
        rs_start(1, 0, 1)
        loads[2] = start_load(2)
        compute_half(1, 0)
        rs_finish(0, 0, 0)
        rs_start(0, 1, 0)
        compute_half(1, 1)
        rs_finish(0, 0, 1)
        rs_start(0, 1, 1)
        loads[3] = start_load(3)
        compute_half(2, 0)
        rs_finish(1, 0, 0)
        rs_start(1, 1, 0)
        compute_half(2, 1)
        rs_finish(1, 0, 1)
        rs_start(1, 1, 1)
        compute_half(3, 0)
        rs_finish(0, 1, 0)
        rs_start(0, 2, 0)
        rs_finish(1, 1, 0)
        rs_start(1, 2, 0)
        compute_half(3, 1)
        rs_finish(0, 1, 1)
        rs_start(0, 2, 1)
        rs_finish(1, 1, 1)
        rs_start(1, 2, 1)

        ag_cw_chunks = [order[2], order[0], order[1]]
        ag_ccw_chunks = [order[1], order[0], order[2]]
        ag = {}
        for sub in range(2):
            rs_finish(0, 2, sub)
            ag[(0, 0, sub)] = ag_copy(0, 0, sub, ag_cw_chunks[0], right)
            ag[(0, 0, sub)].start()
            rs_finish(1, 2, sub)
            ag[(1, 0, sub)] = ag_copy(1, 0, sub, ag_ccw_chunks[0], left)
            ag[(1, 0, sub)].start()
        for t in range(1, N_DEV - 1):
            for sub in range(2):
                ag[(0, t - 1, sub)].wait()
                ag[(0, t, sub)] = ag_copy(0, t, sub, ag_cw_chunks[t], right)
                ag[(0, t, sub)].start()
                ag[(1, t - 1, sub)].wait()
                ag[(1, t, sub)] = ag_copy(1, t, sub, ag_ccw_chunks[t], left)
                ag[(1, t, sub)].start()
        for sub in range(2):
            ag[(0, 2, sub)].wait()
            ag[(1, 2, sub)].wait()

    out = pl.pallas_call(
        body,
        out_shape=jax.ShapeDtypeStruct((B, S, C_out), bf16),
        in_specs=[
            pl.BlockSpec(memory_space=pltpu.VMEM),
            pl.BlockSpec(memory_space=pltpu.VMEM),
            pl.BlockSpec(memory_space=pl.ANY),
        ],
        out_specs=pl.BlockSpec(memory_space=pltpu.VMEM),
        scratch_shapes=[
            pltpu.VMEM((2, S, C), f32),
            pltpu.VMEM((N_DEV - 1, S, H), bf16),
            pltpu.VMEM((N_DEV - 1, S, H), bf16),
            pltpu.SemaphoreType.DMA((2, N_DEV - 1, 2)),
            pltpu.SemaphoreType.DMA((2, N_DEV - 1, 2)),
            pltpu.SemaphoreType.DMA((2, N_DEV - 1, 2)),
            pltpu.SemaphoreType.DMA((2, N_DEV - 1, 2)),
            pltpu.SemaphoreType.DMA((2,)),
        ],
        compiler_params=pltpu.CompilerParams(
            collective_id=0,
            vmem_limit_bytes=100 * 1024 * 1024,
        ),
    )(k, Wp, x)
    return out
